# baseline (speedup 1.0000x reference)
import functools
import math

import jax
import jax.numpy as jnp
from jax import lax
from jax.experimental import pallas as pl
from jax.experimental.pallas import tpu as pltpu
from jax.experimental.pallas import tpu_sc as plsc

N0 = 10000
G = 50
E = 80000
DF = 1036
BIG = 2**30
DPAD = 2048
DPADS = (512, 2048, 2048)
E_PAD = 98304
EROWS = E_PAD // 128
N_PAD = (10240, 2048, 512, 256)
PER = (200, 40, 8)
K = (40, 8, 2)
NPOW2 = (8192, 1024, 256)
NW = 32
_MESH = dict(core_axis_name="c", subcore_axis_name="s", num_cores=2,
             num_subcores=16)


def _sc_edge_pass(n_tab, n_acc, nblk):
  erw = EROWS // NW
  stripe = n_acc // 16
  GP = 1
  for _gp in (6, 4, 3, 2):
    if n_acc * 128 + 16 * (2 * erw * 128 + _gp * 16384) < 2**21:
      GP = _gp
      break

  def body(*refs):
    tabs = refs[:nblk]
    src2d, dst2d, z128 = refs[nblk:nblk + 3]
    s_out = refs[nblk + 3]
    src_v, dst_v, rows_v, acc, sem, sem2 = refs[nblk + 4:]

    c = lax.axis_index("c")
    s = lax.axis_index("s")
    w = c * 16 + s
    pltpu.sync_copy(src2d.at[pl.ds(w * erw, erw)], src_v)
    pltpu.sync_copy(dst2d.at[pl.ds(w * erw, erw)], dst_v)

    for p in range(nblk):
      pltpu.sync_copy(z128.at[pl.ds(s * stripe, stripe)],
                      acc.at[pl.ds(s * stripe, stripe)])
      plsc.subcore_barrier()

      for g in range(erw // GP):
        for j in range(GP):
          r = g * GP + j
          pltpu.async_copy(tabs[p].at[src_v.at[r]], rows_v.at[j], sem)
        for j in range(GP):
          r = g * GP + j
          pltpu.make_async_copy(
              tabs[p].at[src_v.at[r]], rows_v.at[j], sem).wait()
        for j in range(GP):
          r = g * GP + j
          pltpu.async_copy(rows_v.at[j], acc.at[dst_v.at[r]], sem2,
                           add=True)
        for j in range(GP):
          r = g * GP + j
          pltpu.make_async_copy(
              rows_v.at[j], acc.at[dst_v.at[r]], sem2).wait()

      plsc.subcore_barrier()
      pltpu.sync_copy(
          acc.at[pl.ds(s * stripe, stripe)],
          s_out.at[c, pl.ds(p * n_acc + s * stripe, stripe)])
      plsc.subcore_barrier()

  out_type = [jax.ShapeDtypeStruct((2, nblk * n_acc, 128), jnp.float32)]
  scratch = [
      pltpu.VMEM((erw, 128), jnp.int32),
      pltpu.VMEM((erw, 128), jnp.int32),
      pltpu.VMEM((GP, 128, 128), jnp.float32),
      pltpu.VMEM_SHARED((n_acc, 128), jnp.float32),
      pltpu.SemaphoreType.DMA,
      pltpu.SemaphoreType.DMA,
  ]
  return pl.kernel(body, out_type=out_type,
                   mesh=plsc.VectorSubcoreMesh(**_MESH),
                   scratch_types=scratch)


def _sc_gather_relabel(l, n_cur, relabel):
  gk_pad = N_PAD[l + 1]
  b = gk_pad // NW
  erw = EROWS // NW
  npw = NPOW2[l + 1] if l + 1 < len(NPOW2) else 8
  ndum = N_PAD[l + 1]

  def body(x1_hbm, sc16_hbm, perm_hbm, t_hbm, src2d, dst2d,
           xs_out, src_out, dst_out,
           pv, rows_v, scv, t_v, src_v, dst_v, so_v, do_v, sem):
    c = lax.axis_index("c")
    s = lax.axis_index("s")
    w = c * 16 + s
    pltpu.sync_copy(perm_hbm.at[pl.ds(w * b, b)], pv)
    pltpu.async_copy(x1_hbm.at[pv], rows_v, sem).wait()
    pltpu.async_copy(sc16_hbm.at[pv], scv, sem).wait()

    @pl.loop(0, b)
    def _(i):
      sval = scv[i, pl.ds(0, 16)][0]
      for j in range(8):
        rows_v[i, pl.ds(j * 16, 16)] = rows_v[i, pl.ds(j * 16, 16)] * sval

    pltpu.sync_copy(rows_v, xs_out.at[pl.ds(w * b, b)])

    if relabel:
      pltpu.sync_copy(src2d.at[pl.ds(w * erw, erw)], src_v)
      pltpu.sync_copy(dst2d.at[pl.ds(w * erw, erw)], dst_v)
      lanes = lax.iota(jnp.int32, 16)

      GQ = 4
      for gq in range(erw // GQ):
        for j in range(GQ):
          r = gq * GQ + j
          for u in range(8):
            sl = pl.ds(u * 16, 16)
            so_v[r, sl] = jnp.minimum(src_v[r, sl], n_cur - 1)
            do_v[r, sl] = jnp.minimum(dst_v[r, sl], n_cur - 1)
        for j in range(GQ):
          r = gq * GQ + j
          pltpu.async_copy(t_hbm.at[so_v.at[r]], t_v.at[2 * j], sem)
          pltpu.async_copy(t_hbm.at[do_v.at[r]], t_v.at[2 * j + 1], sem)
        for j in range(GQ):
          r = gq * GQ + j
          pltpu.make_async_copy(
              t_hbm.at[so_v.at[r]], t_v.at[2 * j], sem).wait()
          pltpu.make_async_copy(
              t_hbm.at[do_v.at[r]], t_v.at[2 * j + 1], sem).wait()
        for j in range(GQ):
          r = gq * GQ + j
          for u in range(8):
            sl = pl.ds(u * 16, 16)
            s16 = src_v[r, sl]
            d16 = dst_v[r, sl]
            ts = t_v[2 * j, sl]
            td = t_v[2 * j + 1, sl]
            valid = ((s16 < n_cur) & (d16 < n_cur)
                     & (ts < BIG) & (td < BIG))
            e16 = (w * (erw * 128) + r * 128 + u * 16) + lanes
            so_v[r, sl] = jnp.where(valid, ts, e16 & (npw - 1))
            do_v[r, sl] = jnp.where(valid, td, ndum + (e16 & (DPAD - 1)))

      pltpu.sync_copy(so_v, src_out.at[pl.ds(w * erw, erw)])
      pltpu.sync_copy(do_v, dst_out.at[pl.ds(w * erw, erw)])

  out_type = [
      jax.ShapeDtypeStruct((gk_pad, 128), jnp.float32),
      jax.ShapeDtypeStruct((EROWS, 128), jnp.int32),
      jax.ShapeDtypeStruct((EROWS, 128), jnp.int32),
  ]
  scratch = [
      pltpu.VMEM((b,), jnp.int32),
      pltpu.VMEM((b, 128), jnp.float32),
      pltpu.VMEM((b, 128), jnp.float32),
      pltpu.VMEM((8, 128), jnp.int32),
      pltpu.VMEM((erw, 128), jnp.int32),
      pltpu.VMEM((erw, 128), jnp.int32),
      pltpu.VMEM((erw, 128), jnp.int32),
      pltpu.VMEM((erw, 128), jnp.int32),
      pltpu.SemaphoreType.DMA,
  ]
  return pl.kernel(body, out_type=out_type,
                   mesh=plsc.VectorSubcoreMesh(**_MESH),
                   scratch_types=scratch)


def _tc_colmax():
  def body(x_ref, o_ref):
    i = pl.program_id(0)
    m = jnp.max(x_ref[...], axis=0, keepdims=True)
    @pl.when(i == 0)
    def _():
      o_ref[...] = jnp.full_like(o_ref, -jnp.inf)
    o_ref[...] = jnp.maximum(o_ref[...], jnp.broadcast_to(m, (8, 128)))

  return pl.pallas_call(
      body,
      grid=(N0 // 200,),
      in_specs=[pl.BlockSpec((200, 128), lambda i: (i, 0))],
      out_specs=pl.BlockSpec((8, 128), lambda i: (0, 0)),
      out_shape=jax.ShapeDtypeStruct((8, 128), jnp.float32),
  )


def _tc_x1_l0(n_acc):
  nblk = 9
  rb = 256

  def body(agg_ref, xn_ref, wl_ref, wr_ref, bl_ref, o_ref):
    p = pl.program_id(1)
    @pl.when(p == 0)
    def _():
      o_ref[...] = bl_ref[0:1, :] + jnp.dot(
          xn_ref[...], wr_ref[...], preferred_element_type=jnp.float32)
    o_ref[...] += jnp.dot(agg_ref[...], wl_ref[...],
                          preferred_element_type=jnp.float32)
    @pl.when(p == nblk - 1)
    def _():
      o_ref[...] = jnp.maximum(o_ref[...], 0.0)

  blocks_per_seg = n_acc // rb
  return pl.pallas_call(
      body,
      grid=(N_PAD[0] // rb, nblk),
      in_specs=[
          pl.BlockSpec((rb, 128), lambda i, p: (p * blocks_per_seg + i, 0)),
          pl.BlockSpec((rb, DF), lambda i, p: (i, 0)),
          pl.BlockSpec((128, 128), lambda i, p: (p, 0)),
          pl.BlockSpec((DF, 128), lambda i, p: (0, 0)),
          pl.BlockSpec((8, 128), lambda i, p: (0, 0)),
      ],
      out_specs=pl.BlockSpec((rb, 128), lambda i, p: (i, 0)),
      out_shape=jax.ShapeDtypeStruct((N_PAD[0], 128), jnp.float32),
      compiler_params=pltpu.CompilerParams(
          dimension_semantics=("arbitrary", "arbitrary")),
  )


def _tc_x1_l12(n):
  rb = 200

  def body(agg_ref, xs_ref, wl_ref, wr_ref, bl_ref, o_ref):
    o_ref[...] = jnp.maximum(
        jnp.dot(agg_ref[...], wl_ref[...],
                preferred_element_type=jnp.float32)
        + bl_ref[0:1, :]
        + jnp.dot(xs_ref[...], wr_ref[...],
                  preferred_element_type=jnp.float32), 0.0)

  return pl.pallas_call(
      body,
      grid=(n // rb,),
      in_specs=[
          pl.BlockSpec((rb, 128), lambda i: (i, 0)),
          pl.BlockSpec((rb, 128), lambda i: (i, 0)),
          pl.BlockSpec((128, 128), lambda i: (0, 0)),
          pl.BlockSpec((128, 128), lambda i: (0, 0)),
          pl.BlockSpec((8, 128), lambda i: (0, 0)),
      ],
      out_specs=pl.BlockSpec((rb, 128), lambda i: (i, 0)),
      out_shape=jax.ShapeDtypeStruct((n, 128), jnp.float32),
  )


def _tc_v(n):
  rb = 200

  def body(sb_ref, x1_ref, wrel_ref, wroot_ref, brel_ref, o_ref):
    v = (jnp.dot(sb_ref[...], wrel_ref[...],
                 preferred_element_type=jnp.float32)
         + brel_ref[0, 0]
         + jnp.dot(x1_ref[...], wroot_ref[...],
                   preferred_element_type=jnp.float32))
    o_ref[...] = jnp.broadcast_to(v[:, 0:1], (rb, 128))

  return pl.pallas_call(
      body,
      grid=(n // rb,),
      in_specs=[
          pl.BlockSpec((rb, 128), lambda i: (i, 0)),
          pl.BlockSpec((rb, 128), lambda i: (i, 0)),
          pl.BlockSpec((128, 128), lambda i: (0, 0)),
          pl.BlockSpec((128, 128), lambda i: (0, 0)),
          pl.BlockSpec((1, 1), lambda i: (0, 0), memory_space=pltpu.SMEM),
      ],
      out_specs=pl.BlockSpec((rb, 128), lambda i: (i, 0)),
      out_shape=jax.ShapeDtypeStruct((n, 128), jnp.float32),
  )


def _tc_topk(l, n):
  per, k = PER[l], K[l]

  def body(sc_ref, t_ref, p_ref):
    g = pl.program_id(0)
    s_col = sc_ref[:, 0:1]
    rows = lax.broadcasted_iota(jnp.int32, (per, per), 0)
    cols = lax.broadcasted_iota(jnp.int32, (per, per), 1)
    eye = jnp.where(rows == cols, jnp.float32(1), jnp.float32(0))
    s_row = jnp.transpose(s_col)
    gt = s_row > s_col
    tie = (s_row == s_col) & (cols < rows)
    rank_c = jnp.sum(jnp.where(gt | tie, jnp.float32(1), jnp.float32(0)),
                     axis=1, keepdims=True)
    rank_r = lax.dot_general(rank_c, eye, (((0,), (0,)), ((), ())),
                             preferred_element_type=jnp.float32)
    keep_r = rank_r < k
    tv = jnp.where(keep_r, g * k + rank_r.astype(jnp.int32), BIG)
    t_ref[0, 0, :] = tv[0]
    lanes_k = lax.broadcasted_iota(jnp.int32, (per, k), 1)
    onehot = jnp.where(rank_c.astype(jnp.int32) == lanes_k,
                       jnp.float32(1), jnp.float32(0))
    pos = lax.broadcasted_iota(jnp.int32, (per, 1), 0).astype(jnp.float32)
    prow = lax.dot_general(pos, onehot, (((0,), (0,)), ((), ())),
                           preferred_element_type=jnp.float32)
    p_ref[0, 0, :] = (prow[0] + g * per).astype(jnp.int32)

  return pl.pallas_call(
      body,
      grid=(G,),
      in_specs=[pl.BlockSpec((per, 128), lambda g: (g, 0))],
      out_specs=[
          pl.BlockSpec((1, 1, per), lambda g: (g, 0, 0)),
          pl.BlockSpec((1, 1, k), lambda g: (g, 0, 0)),
      ],
      out_shape=[
          jax.ShapeDtypeStruct((G, 1, per), jnp.int32),
          jax.ShapeDtypeStruct((G, 1, k), jnp.int32),
      ],
  )


def _tc_readout(l):
  k = K[l]

  def body(xs_ref, o_ref):
    blk = xs_ref[0]
    gm = jnp.max(blk, axis=0, keepdims=True)
    ga = jnp.sum(blk, axis=0, keepdims=True) / jnp.float32(k)
    o_ref[0] = jnp.concatenate([gm, ga], axis=1)

  return pl.pallas_call(
      body,
      grid=(G,),
      in_specs=[pl.BlockSpec((1, k, 128), lambda g: (g, 0, 0))],
      out_specs=pl.BlockSpec((1, 1, 256), lambda g: (g, 0, 0)),
      out_shape=jax.ShapeDtypeStruct((G, 1, 256), jnp.float32),
  )


def _tc_head():
  def body(h0, h1, h2, w1, b1, w2, b2, wg, bg, wh, bh,
           feat_ref, grade_ref, haz_ref):
    h = h0[...] + h1[...] + h2[...]
    h = jnp.maximum(jnp.dot(h, w1[...], preferred_element_type=jnp.float32)
                    + b1[...], 0.0)
    h = jnp.maximum(jnp.dot(h, w2[...], preferred_element_type=jnp.float32)
                    + b2[...], 0.0)
    rows = lax.broadcasted_iota(jnp.int32, (10, G), 0)
    cols = lax.broadcasted_iota(jnp.int32, (10, G), 1)
    mp = jnp.where(cols // 5 == rows, jnp.float32(1), jnp.float32(0))
    feat = jnp.dot(mp, h, preferred_element_type=jnp.float32) / jnp.float32(5)
    feat_ref[...] = feat
    lg = jnp.dot(feat, wg[...], preferred_element_type=jnp.float32) + bg[...]
    m = jnp.max(lg, axis=1, keepdims=True)
    sh = lg - m
    grade_ref[...] = sh - jnp.log(jnp.sum(jnp.exp(sh), axis=1, keepdims=True))
    hz = jnp.dot(feat, wh[...], preferred_element_type=jnp.float32) + bh[...]
    haz_ref[...] = (1.0 / (1.0 + jnp.exp(-hz))) * 6.0 - 3.0

  full = lambda shp: pl.BlockSpec(shp, lambda: tuple(0 for _ in shp))
  return pl.pallas_call(
      body,
      in_specs=[full((G, 256)), full((G, 256)), full((G, 256)),
                full((256, 128)), full((1, 128)), full((128, 32)),
                full((1, 32)), full((32, 3)), full((1, 3)),
                full((32, 1)), full((1, 1))],
      out_specs=[full((10, 32)), full((10, 3)), full((10, 1))],
      out_shape=[jax.ShapeDtypeStruct((10, 32), jnp.float32),
                 jax.ShapeDtypeStruct((10, 3), jnp.float32),
                 jax.ShapeDtypeStruct((10, 1), jnp.float32)],
  )


def kernel(x, edge_index, edge_attr, batch, pat_idxs, params):
  del edge_attr, batch, pat_idxs
  f32 = jnp.float32
  src, dst = edge_index[0], edge_index[1]
  epad = jnp.arange(E_PAD - E, dtype=jnp.int32)
  src_e = jnp.concatenate([src, epad & (NPOW2[0] - 1)])
  dst_e = jnp.concatenate([dst, N_PAD[0] + (epad & (DPADS[0] - 1))])
  src2d = src_e.reshape(EROWS, 128)
  dst2d = dst_e.reshape(EROWS, 128)

  cm = _tc_colmax()(x[:, :128])
  xn = x.at[:, :12].set(x[:, :12] / cm[0:1, :12])

  hcats = []
  xs_cur = None
  n = N0
  for l in range(3):
    Wl, bl, Wr = params['W_l%d' % l], params['b_l%d' % l], params['W_r%d' % l]
    Wrel, brel = params['W_rel%d' % l], params['b_rel%d' % l]
    Wroot = params['W_root%d' % l]
    npad = N_PAD[l]
    n_acc = npad + DPADS[l]
    z128 = jnp.zeros((n_acc, 128), f32)

    if l == 0:
      xtabs = [jnp.pad(xn[:, 128 * p:128 * (p + 1)],
                       ((0, 0), (0, max(0, 128 * (p + 1) - DF))))
               for p in range(9)]
      ones_tab = jnp.zeros((N0, 128), f32).at[:, 0].set(1.0)
      S2 = _sc_edge_pass(N0, n_acc, 10)(
          *xtabs, ones_tab, src2d, dst2d, z128)[0]
      Sb = S2[0] + S2[1]
      S = Sb[:9 * n_acc]
      cnt = jnp.maximum(Sb[9 * n_acc:, 0:1], 1.0)
      agg = (S.reshape(9, n_acc, 128) / cnt[None]).reshape(9 * n_acc, 128)
      Wlp = jnp.pad(Wl, ((0, 9 * 128 - DF), (0, 0)))
      blr = jnp.broadcast_to(bl[None, :], (8, 128))
      x1p = _tc_x1_l0(n_acc)(agg, xn, Wlp, Wr, blr)
      x1 = x1p[:N0]
    else:
      ones_tab = jnp.zeros((npad, 128), f32).at[:, 0].set(1.0)
      S2 = _sc_edge_pass(npad, n_acc, 2)(
          xs_cur, ones_tab, src2d, dst2d, z128)[0]
      Sb = S2[0] + S2[1]
      cnt = jnp.maximum(Sb[n_acc:n_acc + n, 0:1], 1.0)
      agg = Sb[:n] / cnt
      blr = jnp.broadcast_to(bl[None, :], (8, 128))
      x1 = _tc_x1_l12(n)(agg, xs_cur[:n], Wl, Wr, blr)

    x1t = x1 if n % 8 == 0 else jnp.pad(x1, ((0, 8 - n % 8), (0, 0)))
    SB2 = _sc_edge_pass(x1t.shape[0], n_acc, 1)(
        x1t, src2d, dst2d, z128)[0]
    SB = SB2[0, :n] + SB2[1, :n]
    Wrelp = jnp.pad(Wrel, ((0, 0), (0, 127)))
    Wrootp = jnp.pad(Wroot, ((0, 0), (0, 127)))
    v16 = _tc_v(n)(SB, x1, Wrelp, Wrootp, brel.reshape(1, 1))
    score16 = jnp.tanh(v16)

    t3, p3 = _tc_topk(l, n)(score16)
    t = t3.reshape(-1)
    perm = jnp.pad(p3.reshape(-1), (0, N_PAD[l + 1] - G * K[l]))

    xs_cur, src2d, dst2d = _sc_gather_relabel(l, n, l < 2)(
        x1t, jnp.pad(score16, ((0, x1t.shape[0] - n), (0, 0))), perm, t,
        src2d, dst2d)

    gk = G * K[l]
    hcats.append(_tc_readout(l)(xs_cur[:gk].reshape(G, K[l], 128))
                 .reshape(G, 256))
    n = gk

  b1 = params['enc_b1'].reshape(1, 128)
  b2 = params['enc_b2'].reshape(1, 32)
  bg = params['bg'].reshape(1, 3)
  bh = params['bh'].reshape(1, 1)
  feat, grade, haz = _tc_head()(
      hcats[0], hcats[1], hcats[2], params['enc_W1'], b1,
      params['enc_W2'], b2, params['Wg'], bg, params['Wh'], bh)
  return feat, grade, haz


# speedup vs baseline: 1.0042x; 1.0042x over previous
import functools
import math

import jax
import jax.numpy as jnp
from jax import lax
from jax.experimental import pallas as pl
from jax.experimental.pallas import tpu as pltpu
from jax.experimental.pallas import tpu_sc as plsc

N0 = 10000
G = 50
E = 80000
DF = 1036
BIG = 2**30
DPAD = 2048
DPADS = (512, 2048, 2048)
E_PAD = 98304
EROWS = E_PAD // 128
N_PAD = (10240, 2048, 512, 256)
PER = (200, 40, 8)
K = (40, 8, 2)
NPOW2 = (8192, 1024, 256)
NW = 32
_MESH = dict(core_axis_name="c", subcore_axis_name="s", num_cores=2,
             num_subcores=16)


def _sc_edge_pass(n_tab, n_acc, nblk):
  erw = EROWS // NW
  stripe = n_acc // 16
  GP = 1
  for _gp in (6, 4, 3, 2):
    if n_acc * 128 + 16 * (2 * erw * 128 + _gp * 16384) < 2**21:
      GP = _gp
      break

  def body(*refs):
    tabs = refs[:nblk]
    src2d, dst2d, z128 = refs[nblk:nblk + 3]
    s_out = refs[nblk + 3]
    src_v, dst_v, rows_v, acc, sem, sem2 = refs[nblk + 4:]

    c = lax.axis_index("c")
    s = lax.axis_index("s")
    w = c * 16 + s
    pltpu.sync_copy(src2d.at[pl.ds(w * erw, erw)], src_v)
    pltpu.sync_copy(dst2d.at[pl.ds(w * erw, erw)], dst_v)

    for p in range(nblk):
      pltpu.sync_copy(z128.at[pl.ds(s * stripe, stripe)],
                      acc.at[pl.ds(s * stripe, stripe)])
      plsc.subcore_barrier()

      for g in range(erw // GP):
        for j in range(GP):
          r = g * GP + j
          pltpu.async_copy(tabs[p].at[src_v.at[r]], rows_v.at[j], sem)
        for j in range(GP):
          r = g * GP + j
          pltpu.make_async_copy(
              tabs[p].at[src_v.at[r]], rows_v.at[j], sem).wait()
        for j in range(GP):
          r = g * GP + j
          pltpu.async_copy(rows_v.at[j], acc.at[dst_v.at[r]], sem2,
                           add=True)
        for j in range(GP):
          r = g * GP + j
          pltpu.make_async_copy(
              rows_v.at[j], acc.at[dst_v.at[r]], sem2).wait()

      plsc.subcore_barrier()
      pltpu.sync_copy(
          acc.at[pl.ds(s * stripe, stripe)],
          s_out.at[c, pl.ds(p * n_acc + s * stripe, stripe)])
      plsc.subcore_barrier()

  out_type = [jax.ShapeDtypeStruct((2, nblk * n_acc, 128), jnp.float32)]
  scratch = [
      pltpu.VMEM((erw, 128), jnp.int32),
      pltpu.VMEM((erw, 128), jnp.int32),
      pltpu.VMEM((GP, 128, 128), jnp.float32),
      pltpu.VMEM_SHARED((n_acc, 128), jnp.float32),
      pltpu.SemaphoreType.DMA,
      pltpu.SemaphoreType.DMA,
  ]
  return pl.kernel(body, out_type=out_type,
                   mesh=plsc.VectorSubcoreMesh(**_MESH),
                   scratch_types=scratch)


def _sc_gather_relabel(l, n_cur, relabel):
  gk_pad = N_PAD[l + 1]
  b = gk_pad // NW
  erw = EROWS // NW
  npw = NPOW2[l + 1] if l + 1 < len(NPOW2) else 8
  ndum = N_PAD[l + 1]

  def body(x1_hbm, sc16_hbm, perm_hbm, t_hbm, src2d, dst2d,
           xs_out, src_out, dst_out,
           pv, rows_v, scv, t_v, src_v, dst_v, so_v, do_v, sem):
    c = lax.axis_index("c")
    s = lax.axis_index("s")
    w = c * 16 + s
    pltpu.sync_copy(perm_hbm.at[pl.ds(w * b, b)], pv)
    pltpu.async_copy(x1_hbm.at[pv], rows_v, sem).wait()
    pltpu.async_copy(sc16_hbm.at[pv], scv, sem).wait()

    @pl.loop(0, b)
    def _(i):
      sval = scv[i, pl.ds(0, 16)][0]
      for j in range(8):
        rows_v[i, pl.ds(j * 16, 16)] = rows_v[i, pl.ds(j * 16, 16)] * sval

    pltpu.sync_copy(rows_v, xs_out.at[pl.ds(w * b, b)])

    if relabel:
      pltpu.sync_copy(src2d.at[pl.ds(w * erw, erw)], src_v)
      pltpu.sync_copy(dst2d.at[pl.ds(w * erw, erw)], dst_v)
      lanes = lax.iota(jnp.int32, 16)

      GQ = 4
      for gq in range(erw // GQ):
        for j in range(GQ):
          r = gq * GQ + j
          for u in range(8):
            sl = pl.ds(u * 16, 16)
            so_v[r, sl] = jnp.minimum(src_v[r, sl], n_cur - 1)
            do_v[r, sl] = jnp.minimum(dst_v[r, sl], n_cur - 1)
        for j in range(GQ):
          r = gq * GQ + j
          pltpu.async_copy(t_hbm.at[so_v.at[r]], t_v.at[2 * j], sem)
          pltpu.async_copy(t_hbm.at[do_v.at[r]], t_v.at[2 * j + 1], sem)
        for j in range(GQ):
          r = gq * GQ + j
          pltpu.make_async_copy(
              t_hbm.at[so_v.at[r]], t_v.at[2 * j], sem).wait()
          pltpu.make_async_copy(
              t_hbm.at[do_v.at[r]], t_v.at[2 * j + 1], sem).wait()
        for j in range(GQ):
          r = gq * GQ + j
          for u in range(8):
            sl = pl.ds(u * 16, 16)
            s16 = src_v[r, sl]
            d16 = dst_v[r, sl]
            ts = t_v[2 * j, sl]
            td = t_v[2 * j + 1, sl]
            valid = ((s16 < n_cur) & (d16 < n_cur)
                     & (ts < BIG) & (td < BIG))
            e16 = (w * (erw * 128) + r * 128 + u * 16) + lanes
            so_v[r, sl] = jnp.where(
                valid, ts, s * (npw // 16) + (e16 & (npw // 16 - 1)))
            do_v[r, sl] = jnp.where(
                valid, td, (ndum + s * (DPAD // 16)) + (e16 & (DPAD // 16 - 1)))

      pltpu.sync_copy(so_v, src_out.at[pl.ds(w * erw, erw)])
      pltpu.sync_copy(do_v, dst_out.at[pl.ds(w * erw, erw)])

  out_type = [
      jax.ShapeDtypeStruct((gk_pad, 128), jnp.float32),
      jax.ShapeDtypeStruct((EROWS, 128), jnp.int32),
      jax.ShapeDtypeStruct((EROWS, 128), jnp.int32),
  ]
  scratch = [
      pltpu.VMEM((b,), jnp.int32),
      pltpu.VMEM((b, 128), jnp.float32),
      pltpu.VMEM((b, 128), jnp.float32),
      pltpu.VMEM((8, 128), jnp.int32),
      pltpu.VMEM((erw, 128), jnp.int32),
      pltpu.VMEM((erw, 128), jnp.int32),
      pltpu.VMEM((erw, 128), jnp.int32),
      pltpu.VMEM((erw, 128), jnp.int32),
      pltpu.SemaphoreType.DMA,
  ]
  return pl.kernel(body, out_type=out_type,
                   mesh=plsc.VectorSubcoreMesh(**_MESH),
                   scratch_types=scratch)


def _tc_colmax():
  def body(x_ref, o_ref):
    i = pl.program_id(0)
    m = jnp.max(x_ref[...], axis=0, keepdims=True)
    @pl.when(i == 0)
    def _():
      o_ref[...] = jnp.full_like(o_ref, -jnp.inf)
    o_ref[...] = jnp.maximum(o_ref[...], jnp.broadcast_to(m, (8, 128)))

  return pl.pallas_call(
      body,
      grid=(N0 // 200,),
      in_specs=[pl.BlockSpec((200, 128), lambda i: (i, 0))],
      out_specs=pl.BlockSpec((8, 128), lambda i: (0, 0)),
      out_shape=jax.ShapeDtypeStruct((8, 128), jnp.float32),
  )


def _tc_x1_l0(n_acc):
  nblk = 9
  rb = 256

  def body(agg_ref, xn_ref, wl_ref, wr_ref, bl_ref, o_ref):
    p = pl.program_id(1)
    @pl.when(p == 0)
    def _():
      o_ref[...] = bl_ref[0:1, :] + jnp.dot(
          xn_ref[...], wr_ref[...], preferred_element_type=jnp.float32)
    o_ref[...] += jnp.dot(agg_ref[...], wl_ref[...],
                          preferred_element_type=jnp.float32)
    @pl.when(p == nblk - 1)
    def _():
      o_ref[...] = jnp.maximum(o_ref[...], 0.0)

  blocks_per_seg = n_acc // rb
  return pl.pallas_call(
      body,
      grid=(N_PAD[0] // rb, nblk),
      in_specs=[
          pl.BlockSpec((rb, 128), lambda i, p: (p * blocks_per_seg + i, 0)),
          pl.BlockSpec((rb, DF), lambda i, p: (i, 0)),
          pl.BlockSpec((128, 128), lambda i, p: (p, 0)),
          pl.BlockSpec((DF, 128), lambda i, p: (0, 0)),
          pl.BlockSpec((8, 128), lambda i, p: (0, 0)),
      ],
      out_specs=pl.BlockSpec((rb, 128), lambda i, p: (i, 0)),
      out_shape=jax.ShapeDtypeStruct((N_PAD[0], 128), jnp.float32),
      compiler_params=pltpu.CompilerParams(
          dimension_semantics=("arbitrary", "arbitrary")),
  )


def _tc_x1_l12(n):
  rb = 200

  def body(agg_ref, xs_ref, wl_ref, wr_ref, bl_ref, o_ref):
    o_ref[...] = jnp.maximum(
        jnp.dot(agg_ref[...], wl_ref[...],
                preferred_element_type=jnp.float32)
        + bl_ref[0:1, :]
        + jnp.dot(xs_ref[...], wr_ref[...],
                  preferred_element_type=jnp.float32), 0.0)

  return pl.pallas_call(
      body,
      grid=(n // rb,),
      in_specs=[
          pl.BlockSpec((rb, 128), lambda i: (i, 0)),
          pl.BlockSpec((rb, 128), lambda i: (i, 0)),
          pl.BlockSpec((128, 128), lambda i: (0, 0)),
          pl.BlockSpec((128, 128), lambda i: (0, 0)),
          pl.BlockSpec((8, 128), lambda i: (0, 0)),
      ],
      out_specs=pl.BlockSpec((rb, 128), lambda i: (i, 0)),
      out_shape=jax.ShapeDtypeStruct((n, 128), jnp.float32),
  )


def _tc_v(n):
  rb = 200

  def body(sb_ref, x1_ref, wrel_ref, wroot_ref, brel_ref, o_ref):
    v = (jnp.dot(sb_ref[...], wrel_ref[...],
                 preferred_element_type=jnp.float32)
         + brel_ref[0, 0]
         + jnp.dot(x1_ref[...], wroot_ref[...],
                   preferred_element_type=jnp.float32))
    o_ref[...] = jnp.broadcast_to(v[:, 0:1], (rb, 128))

  return pl.pallas_call(
      body,
      grid=(n // rb,),
      in_specs=[
          pl.BlockSpec((rb, 128), lambda i: (i, 0)),
          pl.BlockSpec((rb, 128), lambda i: (i, 0)),
          pl.BlockSpec((128, 128), lambda i: (0, 0)),
          pl.BlockSpec((128, 128), lambda i: (0, 0)),
          pl.BlockSpec((1, 1), lambda i: (0, 0), memory_space=pltpu.SMEM),
      ],
      out_specs=pl.BlockSpec((rb, 128), lambda i: (i, 0)),
      out_shape=jax.ShapeDtypeStruct((n, 128), jnp.float32),
  )


def _tc_topk(l, n):
  per, k = PER[l], K[l]

  def body(sc_ref, t_ref, p_ref):
    g = pl.program_id(0)
    s_col = sc_ref[:, 0:1]
    rows = lax.broadcasted_iota(jnp.int32, (per, per), 0)
    cols = lax.broadcasted_iota(jnp.int32, (per, per), 1)
    eye = jnp.where(rows == cols, jnp.float32(1), jnp.float32(0))
    s_row = jnp.transpose(s_col)
    gt = s_row > s_col
    tie = (s_row == s_col) & (cols < rows)
    rank_c = jnp.sum(jnp.where(gt | tie, jnp.float32(1), jnp.float32(0)),
                     axis=1, keepdims=True)
    rank_r = lax.dot_general(rank_c, eye, (((0,), (0,)), ((), ())),
                             preferred_element_type=jnp.float32)
    keep_r = rank_r < k
    tv = jnp.where(keep_r, g * k + rank_r.astype(jnp.int32), BIG)
    t_ref[0, 0, :] = tv[0]
    lanes_k = lax.broadcasted_iota(jnp.int32, (per, k), 1)
    onehot = jnp.where(rank_c.astype(jnp.int32) == lanes_k,
                       jnp.float32(1), jnp.float32(0))
    pos = lax.broadcasted_iota(jnp.int32, (per, 1), 0).astype(jnp.float32)
    prow = lax.dot_general(pos, onehot, (((0,), (0,)), ((), ())),
                           preferred_element_type=jnp.float32)
    p_ref[0, 0, :] = (prow[0] + g * per).astype(jnp.int32)

  return pl.pallas_call(
      body,
      grid=(G,),
      in_specs=[pl.BlockSpec((per, 128), lambda g: (g, 0))],
      out_specs=[
          pl.BlockSpec((1, 1, per), lambda g: (g, 0, 0)),
          pl.BlockSpec((1, 1, k), lambda g: (g, 0, 0)),
      ],
      out_shape=[
          jax.ShapeDtypeStruct((G, 1, per), jnp.int32),
          jax.ShapeDtypeStruct((G, 1, k), jnp.int32),
      ],
  )


def _tc_readout(l):
  k = K[l]

  def body(xs_ref, o_ref):
    blk = xs_ref[0]
    gm = jnp.max(blk, axis=0, keepdims=True)
    ga = jnp.sum(blk, axis=0, keepdims=True) / jnp.float32(k)
    o_ref[0] = jnp.concatenate([gm, ga], axis=1)

  return pl.pallas_call(
      body,
      grid=(G,),
      in_specs=[pl.BlockSpec((1, k, 128), lambda g: (g, 0, 0))],
      out_specs=pl.BlockSpec((1, 1, 256), lambda g: (g, 0, 0)),
      out_shape=jax.ShapeDtypeStruct((G, 1, 256), jnp.float32),
  )


def _tc_head():
  def body(h0, h1, h2, w1, b1, w2, b2, wg, bg, wh, bh,
           feat_ref, grade_ref, haz_ref):
    h = h0[...] + h1[...] + h2[...]
    h = jnp.maximum(jnp.dot(h, w1[...], preferred_element_type=jnp.float32)
                    + b1[...], 0.0)
    h = jnp.maximum(jnp.dot(h, w2[...], preferred_element_type=jnp.float32)
                    + b2[...], 0.0)
    rows = lax.broadcasted_iota(jnp.int32, (10, G), 0)
    cols = lax.broadcasted_iota(jnp.int32, (10, G), 1)
    mp = jnp.where(cols // 5 == rows, jnp.float32(1), jnp.float32(0))
    feat = jnp.dot(mp, h, preferred_element_type=jnp.float32) / jnp.float32(5)
    feat_ref[...] = feat
    lg = jnp.dot(feat, wg[...], preferred_element_type=jnp.float32) + bg[...]
    m = jnp.max(lg, axis=1, keepdims=True)
    sh = lg - m
    grade_ref[...] = sh - jnp.log(jnp.sum(jnp.exp(sh), axis=1, keepdims=True))
    hz = jnp.dot(feat, wh[...], preferred_element_type=jnp.float32) + bh[...]
    haz_ref[...] = (1.0 / (1.0 + jnp.exp(-hz))) * 6.0 - 3.0

  full = lambda shp: pl.BlockSpec(shp, lambda: tuple(0 for _ in shp))
  return pl.pallas_call(
      body,
      in_specs=[full((G, 256)), full((G, 256)), full((G, 256)),
                full((256, 128)), full((1, 128)), full((128, 32)),
                full((1, 32)), full((32, 3)), full((1, 3)),
                full((32, 1)), full((1, 1))],
      out_specs=[full((10, 32)), full((10, 3)), full((10, 1))],
      out_shape=[jax.ShapeDtypeStruct((10, 32), jnp.float32),
                 jax.ShapeDtypeStruct((10, 3), jnp.float32),
                 jax.ShapeDtypeStruct((10, 1), jnp.float32)],
  )


def kernel(x, edge_index, edge_attr, batch, pat_idxs, params):
  del edge_attr, batch, pat_idxs
  f32 = jnp.float32
  src, dst = edge_index[0], edge_index[1]
  epad = jnp.arange(E_PAD - E, dtype=jnp.int32)
  spad = ((E + epad) // (E_PAD // NW)) % 16
  src_e = jnp.concatenate(
      [src, spad * (NPOW2[0] // 16) + (epad & (NPOW2[0] // 16 - 1))])
  dst_e = jnp.concatenate(
      [dst, (N_PAD[0] + spad * (DPADS[0] // 16))
            + (epad & (DPADS[0] // 16 - 1))])
  src2d = src_e.reshape(EROWS, 128)
  dst2d = dst_e.reshape(EROWS, 128)

  cm = _tc_colmax()(x[:, :128])
  xn = x.at[:, :12].set(x[:, :12] / cm[0:1, :12])

  hcats = []
  xs_cur = None
  n = N0
  for l in range(3):
    Wl, bl, Wr = params['W_l%d' % l], params['b_l%d' % l], params['W_r%d' % l]
    Wrel, brel = params['W_rel%d' % l], params['b_rel%d' % l]
    Wroot = params['W_root%d' % l]
    npad = N_PAD[l]
    n_acc = npad + DPADS[l]
    z128 = jnp.zeros((n_acc, 128), f32)

    if l == 0:
      xtabs = [jnp.pad(xn[:, 128 * p:128 * (p + 1)],
                       ((0, 0), (0, max(0, 128 * (p + 1) - DF))))
               for p in range(9)]
      ones_tab = jnp.zeros((N0, 128), f32).at[:, 0].set(1.0)
      S2 = _sc_edge_pass(N0, n_acc, 10)(
          *xtabs, ones_tab, src2d, dst2d, z128)[0]
      Sb = S2[0] + S2[1]
      S = Sb[:9 * n_acc]
      cnt = jnp.maximum(Sb[9 * n_acc:, 0:1], 1.0)
      agg = (S.reshape(9, n_acc, 128) / cnt[None]).reshape(9 * n_acc, 128)
      Wlp = jnp.pad(Wl, ((0, 9 * 128 - DF), (0, 0)))
      blr = jnp.broadcast_to(bl[None, :], (8, 128))
      x1p = _tc_x1_l0(n_acc)(agg, xn, Wlp, Wr, blr)
      x1 = x1p[:N0]
    else:
      ones_tab = jnp.zeros((npad, 128), f32).at[:, 0].set(1.0)
      S2 = _sc_edge_pass(npad, n_acc, 2)(
          xs_cur, ones_tab, src2d, dst2d, z128)[0]
      Sb = S2[0] + S2[1]
      cnt = jnp.maximum(Sb[n_acc:n_acc + n, 0:1], 1.0)
      agg = Sb[:n] / cnt
      blr = jnp.broadcast_to(bl[None, :], (8, 128))
      x1 = _tc_x1_l12(n)(agg, xs_cur[:n], Wl, Wr, blr)

    x1t = x1 if n % 8 == 0 else jnp.pad(x1, ((0, 8 - n % 8), (0, 0)))
    SB2 = _sc_edge_pass(x1t.shape[0], n_acc, 1)(
        x1t, src2d, dst2d, z128)[0]
    SB = SB2[0, :n] + SB2[1, :n]
    Wrelp = jnp.pad(Wrel, ((0, 0), (0, 127)))
    Wrootp = jnp.pad(Wroot, ((0, 0), (0, 127)))
    v16 = _tc_v(n)(SB, x1, Wrelp, Wrootp, brel.reshape(1, 1))
    score16 = jnp.tanh(v16)

    t3, p3 = _tc_topk(l, n)(score16)
    t = t3.reshape(-1)
    perm = jnp.pad(p3.reshape(-1), (0, N_PAD[l + 1] - G * K[l]))

    xs_cur, src2d, dst2d = _sc_gather_relabel(l, n, l < 2)(
        x1t, jnp.pad(score16, ((0, x1t.shape[0] - n), (0, 0))), perm, t,
        src2d, dst2d)

    gk = G * K[l]
    hcats.append(_tc_readout(l)(xs_cur[:gk].reshape(G, K[l], 128))
                 .reshape(G, 256))
    n = gk

  b1 = params['enc_b1'].reshape(1, 128)
  b2 = params['enc_b2'].reshape(1, 32)
  bg = params['bg'].reshape(1, 3)
  bh = params['bh'].reshape(1, 1)
  feat, grade, haz = _tc_head()(
      hcats[0], hcats[1], hcats[2], params['enc_W1'], b1,
      params['enc_W2'], b2, params['Wg'], bg, params['Wh'], bh)
  return feat, grade, haz


# speedup vs baseline: 1.0048x; 1.0006x over previous
import jax
import jax.numpy as jnp
from jax import lax
from jax.experimental import pallas as pl
from jax.experimental.pallas import tpu as pltpu
from jax.experimental.pallas import tpu_sc as plsc

N0 = 10000
G = 50
E = 80000
DF = 1036
BIG = 2**30
DPAD = 2048
DPADS = (512, 2048, 2048)
E_PAD = 98304
EROWS = E_PAD // 128
N_PAD = (10240, 2048, 512, 256)
PER = (200, 40, 8)
K = (40, 8, 2)
NPOW2 = (8192, 1024, 256)
NW = 32
_MESH = dict(core_axis_name="c", subcore_axis_name="s", num_cores=2,
             num_subcores=16)


def _sc_edge_pass(n_tab, n_acc, nblk):
  erw = EROWS // NW
  stripe = n_acc // 16
  GP = 1
  for _gp in (6, 4, 3, 2):
    if n_acc * 128 + 16 * (2 * erw * 128 + _gp * 16384) < 2**21:
      GP = _gp
      break

  def body(*refs):
    tabs = refs[:nblk]
    src2d, dst2d, z128 = refs[nblk:nblk + 3]
    s_out = refs[nblk + 3]
    src_v, dst_v, rows_v, acc, sem, sem2 = refs[nblk + 4:]

    c = lax.axis_index("c")
    s = lax.axis_index("s")
    w = c * 16 + s
    pltpu.sync_copy(src2d.at[pl.ds(w * erw, erw)], src_v)
    pltpu.sync_copy(dst2d.at[pl.ds(w * erw, erw)], dst_v)

    for p in range(nblk):
      pltpu.sync_copy(z128.at[pl.ds(s * stripe, stripe)],
                      acc.at[pl.ds(s * stripe, stripe)])
      plsc.subcore_barrier()

      for g in range(erw // GP):
        for j in range(GP):
          r = g * GP + j
          pltpu.async_copy(tabs[p].at[src_v.at[r]], rows_v.at[j], sem)
        for j in range(GP):
          r = g * GP + j
          pltpu.make_async_copy(
              tabs[p].at[src_v.at[r]], rows_v.at[j], sem).wait()
        for j in range(GP):
          r = g * GP + j
          pltpu.async_copy(rows_v.at[j], acc.at[dst_v.at[r]], sem2,
                           add=True)
        for j in range(GP):
          r = g * GP + j
          pltpu.make_async_copy(
              rows_v.at[j], acc.at[dst_v.at[r]], sem2).wait()

      plsc.subcore_barrier()
      pltpu.sync_copy(
          acc.at[pl.ds(s * stripe, stripe)],
          s_out.at[c, pl.ds(p * n_acc + s * stripe, stripe)])
      plsc.subcore_barrier()

  out_type = [jax.ShapeDtypeStruct((2, nblk * n_acc, 128), jnp.float32)]
  scratch = [
      pltpu.VMEM((erw, 128), jnp.int32),
      pltpu.VMEM((erw, 128), jnp.int32),
      pltpu.VMEM((GP, 128, 128), jnp.float32),
      pltpu.VMEM_SHARED((n_acc, 128), jnp.float32),
      pltpu.SemaphoreType.DMA,
      pltpu.SemaphoreType.DMA,
  ]
  return pl.kernel(body, out_type=out_type,
                   mesh=plsc.VectorSubcoreMesh(**_MESH),
                   scratch_types=scratch)


def _sc_gather_relabel(l, n_cur, relabel):
  gk_pad = N_PAD[l + 1]
  b = gk_pad // NW
  erw = EROWS // NW
  npw = NPOW2[l + 1] if l + 1 < len(NPOW2) else 8
  ndum = N_PAD[l + 1]

  def body(x1_hbm, sc16_hbm, perm_hbm, t_hbm, src2d, dst2d,
           xs_out, src_out, dst_out,
           pv, rows_v, scv, t_v, src_v, dst_v, so_v, do_v, sem):
    c = lax.axis_index("c")
    s = lax.axis_index("s")
    w = c * 16 + s
    pltpu.sync_copy(perm_hbm.at[pl.ds(w * b, b)], pv)
    pltpu.async_copy(x1_hbm.at[pv], rows_v, sem).wait()
    pltpu.async_copy(sc16_hbm.at[pv], scv, sem).wait()

    @pl.loop(0, b)
    def _(i):
      sval = scv[i, pl.ds(0, 16)][0]
      for j in range(8):
        rows_v[i, pl.ds(j * 16, 16)] = rows_v[i, pl.ds(j * 16, 16)] * sval

    pltpu.sync_copy(rows_v, xs_out.at[pl.ds(w * b, b)])

    if relabel:
      pltpu.sync_copy(src2d.at[pl.ds(w * erw, erw)], src_v)
      pltpu.sync_copy(dst2d.at[pl.ds(w * erw, erw)], dst_v)
      lanes = lax.iota(jnp.int32, 16)

      GQ = 4
      for gq in range(erw // GQ):
        for j in range(GQ):
          r = gq * GQ + j
          for u in range(8):
            sl = pl.ds(u * 16, 16)
            so_v[r, sl] = jnp.minimum(src_v[r, sl], n_cur - 1)
            do_v[r, sl] = jnp.minimum(dst_v[r, sl], n_cur - 1)
        for j in range(GQ):
          r = gq * GQ + j
          pltpu.async_copy(t_hbm.at[so_v.at[r]], t_v.at[2 * j], sem)
          pltpu.async_copy(t_hbm.at[do_v.at[r]], t_v.at[2 * j + 1], sem)
        for j in range(GQ):
          r = gq * GQ + j
          pltpu.make_async_copy(
              t_hbm.at[so_v.at[r]], t_v.at[2 * j], sem).wait()
          pltpu.make_async_copy(
              t_hbm.at[do_v.at[r]], t_v.at[2 * j + 1], sem).wait()
        for j in range(GQ):
          r = gq * GQ + j
          for u in range(8):
            sl = pl.ds(u * 16, 16)
            s16 = src_v[r, sl]
            d16 = dst_v[r, sl]
            ts = t_v[2 * j, sl]
            td = t_v[2 * j + 1, sl]
            valid = ((s16 < n_cur) & (d16 < n_cur)
                     & (ts < BIG) & (td < BIG))
            e16 = (w * (erw * 128) + r * 128 + u * 16) + lanes
            so_v[r, sl] = jnp.where(
                valid, ts, s * (npw // 16) + (e16 & (npw // 16 - 1)))
            do_v[r, sl] = jnp.where(
                valid, td, (ndum + s * (DPAD // 16)) + (e16 & (DPAD // 16 - 1)))

      pltpu.sync_copy(so_v, src_out.at[pl.ds(w * erw, erw)])
      pltpu.sync_copy(do_v, dst_out.at[pl.ds(w * erw, erw)])

  out_type = [
      jax.ShapeDtypeStruct((gk_pad, 128), jnp.float32),
      jax.ShapeDtypeStruct((EROWS, 128), jnp.int32),
      jax.ShapeDtypeStruct((EROWS, 128), jnp.int32),
  ]
  scratch = [
      pltpu.VMEM((b,), jnp.int32),
      pltpu.VMEM((b, 128), jnp.float32),
      pltpu.VMEM((b, 128), jnp.float32),
      pltpu.VMEM((8, 128), jnp.int32),
      pltpu.VMEM((erw, 128), jnp.int32),
      pltpu.VMEM((erw, 128), jnp.int32),
      pltpu.VMEM((erw, 128), jnp.int32),
      pltpu.VMEM((erw, 128), jnp.int32),
      pltpu.SemaphoreType.DMA,
  ]
  return pl.kernel(body, out_type=out_type,
                   mesh=plsc.VectorSubcoreMesh(**_MESH),
                   scratch_types=scratch)


def _tc_colmax():
  def body(x_ref, o_ref):
    i = pl.program_id(0)
    m = jnp.max(x_ref[...], axis=0, keepdims=True)
    @pl.when(i == 0)
    def _():
      o_ref[...] = jnp.full_like(o_ref, -jnp.inf)
    o_ref[...] = jnp.maximum(o_ref[...], jnp.broadcast_to(m, (8, 128)))

  return pl.pallas_call(
      body,
      grid=(N0 // 200,),
      in_specs=[pl.BlockSpec((200, 128), lambda i: (i, 0))],
      out_specs=pl.BlockSpec((8, 128), lambda i: (0, 0)),
      out_shape=jax.ShapeDtypeStruct((8, 128), jnp.float32),
  )


def _tc_x1_l0(n_acc):
  nblk = 9
  rb = 256

  def body(agg_ref, xn_ref, wl_ref, wr_ref, bl_ref, o_ref):
    p = pl.program_id(1)
    @pl.when(p == 0)
    def _():
      o_ref[...] = bl_ref[0:1, :] + jnp.dot(
          xn_ref[...], wr_ref[...], preferred_element_type=jnp.float32)
    o_ref[...] += jnp.dot(agg_ref[...], wl_ref[...],
                          preferred_element_type=jnp.float32)
    @pl.when(p == nblk - 1)
    def _():
      o_ref[...] = jnp.maximum(o_ref[...], 0.0)

  blocks_per_seg = n_acc // rb
  return pl.pallas_call(
      body,
      grid=(N_PAD[0] // rb, nblk),
      in_specs=[
          pl.BlockSpec((rb, 128), lambda i, p: (p * blocks_per_seg + i, 0)),
          pl.BlockSpec((rb, DF), lambda i, p: (i, 0)),
          pl.BlockSpec((128, 128), lambda i, p: (p, 0)),
          pl.BlockSpec((DF, 128), lambda i, p: (0, 0)),
          pl.BlockSpec((8, 128), lambda i, p: (0, 0)),
      ],
      out_specs=pl.BlockSpec((rb, 128), lambda i, p: (i, 0)),
      out_shape=jax.ShapeDtypeStruct((N_PAD[0], 128), jnp.float32),
      compiler_params=pltpu.CompilerParams(
          dimension_semantics=("arbitrary", "arbitrary")),
  )


def _tc_x1_l12(n):
  rb = 200

  def body(agg_ref, xs_ref, wl_ref, wr_ref, bl_ref, o_ref):
    o_ref[...] = jnp.maximum(
        jnp.dot(agg_ref[...], wl_ref[...],
                preferred_element_type=jnp.float32)
        + bl_ref[0:1, :]
        + jnp.dot(xs_ref[...], wr_ref[...],
                  preferred_element_type=jnp.float32), 0.0)

  return pl.pallas_call(
      body,
      grid=(n // rb,),
      in_specs=[
          pl.BlockSpec((rb, 128), lambda i: (i, 0)),
          pl.BlockSpec((rb, 128), lambda i: (i, 0)),
          pl.BlockSpec((128, 128), lambda i: (0, 0)),
          pl.BlockSpec((128, 128), lambda i: (0, 0)),
          pl.BlockSpec((8, 128), lambda i: (0, 0)),
      ],
      out_specs=pl.BlockSpec((rb, 128), lambda i: (i, 0)),
      out_shape=jax.ShapeDtypeStruct((n, 128), jnp.float32),
  )


def _tc_v(n):
  rb = 200

  def body(sb_ref, x1_ref, wrel_ref, wroot_ref, brel_ref, o_ref):
    v = (jnp.dot(sb_ref[...], wrel_ref[...],
                 preferred_element_type=jnp.float32)
         + brel_ref[0, 0]
         + jnp.dot(x1_ref[...], wroot_ref[...],
                   preferred_element_type=jnp.float32))
    o_ref[...] = jnp.broadcast_to(v[:, 0:1], (rb, 128))

  return pl.pallas_call(
      body,
      grid=(n // rb,),
      in_specs=[
          pl.BlockSpec((rb, 128), lambda i: (i, 0)),
          pl.BlockSpec((rb, 128), lambda i: (i, 0)),
          pl.BlockSpec((128, 128), lambda i: (0, 0)),
          pl.BlockSpec((128, 128), lambda i: (0, 0)),
          pl.BlockSpec((1, 1), lambda i: (0, 0), memory_space=pltpu.SMEM),
      ],
      out_specs=pl.BlockSpec((rb, 128), lambda i: (i, 0)),
      out_shape=jax.ShapeDtypeStruct((n, 128), jnp.float32),
  )


def _tc_topk(l, n):
  per, k = PER[l], K[l]

  def body(sc_ref, t_ref, p_ref):
    g = pl.program_id(0)
    s_col = sc_ref[:, 0:1]
    rows = lax.broadcasted_iota(jnp.int32, (per, per), 0)
    cols = lax.broadcasted_iota(jnp.int32, (per, per), 1)
    eye = jnp.where(rows == cols, jnp.float32(1), jnp.float32(0))
    s_row = jnp.transpose(s_col)
    gt = s_row > s_col
    tie = (s_row == s_col) & (cols < rows)
    rank_c = jnp.sum(jnp.where(gt | tie, jnp.float32(1), jnp.float32(0)),
                     axis=1, keepdims=True)
    rank_r = lax.dot_general(rank_c, eye, (((0,), (0,)), ((), ())),
                             preferred_element_type=jnp.float32)
    keep_r = rank_r < k
    tv = jnp.where(keep_r, g * k + rank_r.astype(jnp.int32), BIG)
    t_ref[0, 0, :] = tv[0]
    lanes_k = lax.broadcasted_iota(jnp.int32, (per, k), 1)
    onehot = jnp.where(rank_c.astype(jnp.int32) == lanes_k,
                       jnp.float32(1), jnp.float32(0))
    pos = lax.broadcasted_iota(jnp.int32, (per, 1), 0).astype(jnp.float32)
    prow = lax.dot_general(pos, onehot, (((0,), (0,)), ((), ())),
                           preferred_element_type=jnp.float32)
    p_ref[0, 0, :] = (prow[0] + g * per).astype(jnp.int32)

  return pl.pallas_call(
      body,
      grid=(G,),
      in_specs=[pl.BlockSpec((per, 128), lambda g: (g, 0))],
      out_specs=[
          pl.BlockSpec((1, 1, per), lambda g: (g, 0, 0)),
          pl.BlockSpec((1, 1, k), lambda g: (g, 0, 0)),
      ],
      out_shape=[
          jax.ShapeDtypeStruct((G, 1, per), jnp.int32),
          jax.ShapeDtypeStruct((G, 1, k), jnp.int32),
      ],
  )


def _tc_readout(l):
  k = K[l]

  def body(xs_ref, o_ref):
    blk = xs_ref[0]
    gm = jnp.max(blk, axis=0, keepdims=True)
    ga = jnp.sum(blk, axis=0, keepdims=True) / jnp.float32(k)
    o_ref[0] = jnp.concatenate([gm, ga], axis=1)

  return pl.pallas_call(
      body,
      grid=(G,),
      in_specs=[pl.BlockSpec((1, k, 128), lambda g: (g, 0, 0))],
      out_specs=pl.BlockSpec((1, 1, 256), lambda g: (g, 0, 0)),
      out_shape=jax.ShapeDtypeStruct((G, 1, 256), jnp.float32),
  )


def _tc_head():
  def body(h0, h1, h2, w1, b1, w2, b2, wg, bg, wh, bh,
           feat_ref, grade_ref, haz_ref):
    h = h0[...] + h1[...] + h2[...]
    h = jnp.maximum(jnp.dot(h, w1[...], preferred_element_type=jnp.float32)
                    + b1[...], 0.0)
    h = jnp.maximum(jnp.dot(h, w2[...], preferred_element_type=jnp.float32)
                    + b2[...], 0.0)
    rows = lax.broadcasted_iota(jnp.int32, (10, G), 0)
    cols = lax.broadcasted_iota(jnp.int32, (10, G), 1)
    mp = jnp.where(cols // 5 == rows, jnp.float32(1), jnp.float32(0))
    feat = jnp.dot(mp, h, preferred_element_type=jnp.float32) / jnp.float32(5)
    feat_ref[...] = feat
    lg = jnp.dot(feat, wg[...], preferred_element_type=jnp.float32) + bg[...]
    m = jnp.max(lg, axis=1, keepdims=True)
    sh = lg - m
    grade_ref[...] = sh - jnp.log(jnp.sum(jnp.exp(sh), axis=1, keepdims=True))
    hz = jnp.dot(feat, wh[...], preferred_element_type=jnp.float32) + bh[...]
    haz_ref[...] = (1.0 / (1.0 + jnp.exp(-hz))) * 6.0 - 3.0

  full = lambda shp: pl.BlockSpec(shp, lambda: tuple(0 for _ in shp))
  return pl.pallas_call(
      body,
      in_specs=[full((G, 256)), full((G, 256)), full((G, 256)),
                full((256, 128)), full((1, 128)), full((128, 32)),
                full((1, 32)), full((32, 3)), full((1, 3)),
                full((32, 1)), full((1, 1))],
      out_specs=[full((10, 32)), full((10, 3)), full((10, 1))],
      out_shape=[jax.ShapeDtypeStruct((10, 32), jnp.float32),
                 jax.ShapeDtypeStruct((10, 3), jnp.float32),
                 jax.ShapeDtypeStruct((10, 1), jnp.float32)],
  )


def kernel(x, edge_index, edge_attr, batch, pat_idxs, params):
  del edge_attr, batch, pat_idxs
  f32 = jnp.float32
  src, dst = edge_index[0], edge_index[1]
  epad = jnp.arange(E_PAD - E, dtype=jnp.int32)
  spad = ((E + epad) // (E_PAD // NW)) % 16
  src_e = jnp.concatenate(
      [src, spad * (NPOW2[0] // 16) + (epad & (NPOW2[0] // 16 - 1))])
  dst_e = jnp.concatenate(
      [dst, (N_PAD[0] + spad * (DPADS[0] // 16))
            + (epad & (DPADS[0] // 16 - 1))])
  src2d = src_e.reshape(EROWS, 128)
  dst2d = dst_e.reshape(EROWS, 128)

  cm = _tc_colmax()(x[:, :128])
  xn = x.at[:, :12].set(x[:, :12] / cm[0:1, :12])

  hcats = []
  xs_cur = None
  n = N0
  for l in range(3):
    Wl, bl, Wr = params['W_l%d' % l], params['b_l%d' % l], params['W_r%d' % l]
    Wrel, brel = params['W_rel%d' % l], params['b_rel%d' % l]
    Wroot = params['W_root%d' % l]
    npad = N_PAD[l]
    n_acc = npad + DPADS[l]
    z128 = jnp.zeros((n_acc, 128), f32)

    if l == 0:
      xtabs = [jnp.pad(xn[:, 128 * p:128 * (p + 1)],
                       ((0, 0), (0, max(0, 128 * (p + 1) - DF))))
               for p in range(9)]
      ones_tab = jnp.zeros((N0, 128), f32).at[:, 0].set(1.0)
      S2 = _sc_edge_pass(N0, n_acc, 10)(
          *xtabs, ones_tab, src2d, dst2d, z128)[0]
      Sb = S2[0] + S2[1]
      S = Sb[:9 * n_acc]
      cnt = jnp.maximum(Sb[9 * n_acc:, 0:1], 1.0)
      agg = (S.reshape(9, n_acc, 128) / cnt[None]).reshape(9 * n_acc, 128)
      Wlp = jnp.pad(Wl, ((0, 9 * 128 - DF), (0, 0)))
      blr = jnp.broadcast_to(bl[None, :], (8, 128))
      x1p = _tc_x1_l0(n_acc)(agg, xn, Wlp, Wr, blr)
      x1 = x1p[:N0]
    else:
      ones_tab = jnp.zeros((npad, 128), f32).at[:, 0].set(1.0)
      S2 = _sc_edge_pass(npad, n_acc, 2)(
          xs_cur, ones_tab, src2d, dst2d, z128)[0]
      Sb = S2[0] + S2[1]
      cnt = jnp.maximum(Sb[n_acc:n_acc + n, 0:1], 1.0)
      agg = Sb[:n] / cnt
      blr = jnp.broadcast_to(bl[None, :], (8, 128))
      x1 = _tc_x1_l12(n)(agg, xs_cur[:n], Wl, Wr, blr)

    x1t = x1 if n % 8 == 0 else jnp.pad(x1, ((0, 8 - n % 8), (0, 0)))
    SB2 = _sc_edge_pass(x1t.shape[0], n_acc, 1)(
        x1t, src2d, dst2d, z128)[0]
    SB = SB2[0, :n] + SB2[1, :n]
    Wrelp = jnp.pad(Wrel, ((0, 0), (0, 127)))
    Wrootp = jnp.pad(Wroot, ((0, 0), (0, 127)))
    v16 = _tc_v(n)(SB, x1, Wrelp, Wrootp, brel.reshape(1, 1))
    score16 = jnp.tanh(v16)

    t3, p3 = _tc_topk(l, n)(score16)
    t = t3.reshape(-1)
    perm = jnp.pad(p3.reshape(-1), (0, N_PAD[l + 1] - G * K[l]))

    xs_cur, src2d, dst2d = _sc_gather_relabel(l, n, l < 2)(
        x1t, jnp.pad(score16, ((0, x1t.shape[0] - n), (0, 0))), perm, t,
        src2d, dst2d)

    gk = G * K[l]
    hcats.append(_tc_readout(l)(xs_cur[:gk].reshape(G, K[l], 128))
                 .reshape(G, 256))
    n = gk

  b1 = params['enc_b1'].reshape(1, 128)
  b2 = params['enc_b2'].reshape(1, 32)
  bg = params['bg'].reshape(1, 3)
  bh = params['bh'].reshape(1, 1)
  feat, grade, haz = _tc_head()(
      hcats[0], hcats[1], hcats[2], params['enc_W1'], b1,
      params['enc_W2'], b2, params['Wg'], bg, params['Wh'], bh)
  return feat, grade, haz
